# Initial kernel scaffold; baseline (speedup 1.0000x reference)
import jax
import jax.numpy as jnp
from jax.experimental import pallas as pl


def kernel(inputs, codebook):
    raise NotImplementedError("write your pallas kernel here")


# speedup vs baseline: 1.3145x; 1.3145x over previous
import jax
import jax.numpy as jnp
from jax import lax
from jax.experimental import pallas as pl

_NUM_EMB = 1024
_COMMIT = 0.25
_EPS = 1e-10


def _vq_body(x_ref, cb_ref, out_ref, loss_ref, perp_ref, hist_ref):
    i = pl.program_id(0)
    nb = pl.num_programs(0)

    @pl.when(i == 0)
    def _init():
        loss_ref[...] = jnp.zeros_like(loss_ref)
        hist_ref[...] = jnp.zeros_like(hist_ref)
        perp_ref[...] = jnp.zeros_like(perp_ref)

    x_b = x_ref[0]
    xt = jnp.transpose(x_b, (1, 0))
    cb = cb_ref[...]

    xsq = jnp.sum(xt * xt, axis=1, keepdims=True)
    cbsq = jnp.sum(cb * cb, axis=1)
    mm = lax.dot_general(xt, cb, (((1,), (1,)), ((), ())),
                         preferred_element_type=jnp.float32)
    dist = (xsq + cbsq[None, :]) - 2.0 * mm

    min_d = jnp.min(dist, axis=1, keepdims=True)
    cols = lax.broadcasted_iota(jnp.int32, dist.shape, 1)
    idx = jnp.min(jnp.where(dist == min_d, cols, _NUM_EMB), axis=1)
    onehot = (cols == idx[:, None]).astype(jnp.float32)
    q = jnp.dot(onehot, cb, preferred_element_type=jnp.float32)
    out_ref[0] = jnp.transpose(q, (1, 0))

    loss_ref[...] += jnp.sum(min_d, axis=0, keepdims=True)
    hist_ref[...] += jnp.sum(onehot, axis=0, keepdims=True)

    @pl.when(i == nb - 1)
    def _final():
        n_tok = jnp.float32(nb * xt.shape[0])
        total = n_tok * jnp.float32(xt.shape[1])
        loss_ref[...] = (1.0 + _COMMIT) * loss_ref[...] / total
        p = hist_ref[...] / n_tok
        ent = jnp.sum(p * jnp.log(p + _EPS), axis=1, keepdims=True)
        perp_ref[...] = jnp.exp(-ent)


def kernel(inputs, codebook):
    b, c, n = inputs.shape
    out_q, loss, perp, _hist = pl.pallas_call(
        _vq_body,
        grid=(b,),
        in_specs=[
            pl.BlockSpec((1, c, n), lambda i: (i, 0, 0)),
            pl.BlockSpec((_NUM_EMB, c), lambda i: (0, 0)),
        ],
        out_specs=[
            pl.BlockSpec((1, c, n), lambda i: (i, 0, 0)),
            pl.BlockSpec((1, 1), lambda i: (0, 0)),
            pl.BlockSpec((1, 1), lambda i: (0, 0)),
            pl.BlockSpec((1, _NUM_EMB), lambda i: (0, 0)),
        ],
        out_shape=[
            jax.ShapeDtypeStruct((b, c, n), jnp.float32),
            jax.ShapeDtypeStruct((1, 1), jnp.float32),
            jax.ShapeDtypeStruct((1, 1), jnp.float32),
            jax.ShapeDtypeStruct((1, _NUM_EMB), jnp.float32),
        ],
    )(inputs, codebook)
    return (loss[0, 0], out_q, perp[0, 0])


# speedup vs baseline: 1.5965x; 1.2145x over previous
import jax
import jax.numpy as jnp
from jax import lax
from jax.experimental import pallas as pl

_NUM_EMB = 1024
_COMMIT = 0.25
_EPS = 1e-10


def _vq_body(x_ref, cb_ref, out_ref, loss_ref, perp_ref, hist_ref):
    i = pl.program_id(0)
    nb = pl.num_programs(0)

    @pl.when(i == 0)
    def _init():
        loss_ref[...] = jnp.zeros_like(loss_ref)
        hist_ref[...] = jnp.zeros_like(hist_ref)
        perp_ref[...] = jnp.zeros_like(perp_ref)

    x_b = x_ref[0]
    xt = jnp.transpose(x_b, (1, 0))
    cb = cb_ref[...]

    xsq = jnp.sum(xt * xt, axis=1, keepdims=True)
    cbsq = jnp.sum(cb * cb, axis=1)
    mm = lax.dot_general(xt, cb, (((1,), (1,)), ((), ())),
                         preferred_element_type=jnp.float32)
    dist = (xsq + cbsq[None, :]) - 2.0 * mm

    min_d = jnp.min(dist, axis=1, keepdims=True)
    cols = lax.broadcasted_iota(jnp.int32, dist.shape, 1)
    idx = jnp.min(jnp.where(dist == min_d, cols, _NUM_EMB), axis=1)
    rows_cm = lax.broadcasted_iota(jnp.int32, (_NUM_EMB, xt.shape[0]), 0)
    onehot_cm = (rows_cm == idx[None, :]).astype(jnp.float32)
    qt = lax.dot_general(cb, onehot_cm, (((0,), (0,)), ((), ())),
                         preferred_element_type=jnp.float32)
    out_ref[0] = qt

    loss_ref[...] += jnp.sum(min_d, axis=0, keepdims=True)
    hist_ref[...] += jnp.sum(onehot_cm, axis=1, keepdims=True)

    @pl.when(i == nb - 1)
    def _final():
        n_tok = jnp.float32(nb * xt.shape[0])
        total = n_tok * jnp.float32(xt.shape[1])
        loss_ref[...] = (1.0 + _COMMIT) * loss_ref[...] / total
        p = hist_ref[...] / n_tok
        ent = jnp.sum(p * jnp.log(p + _EPS), axis=0, keepdims=True)
        perp_ref[...] = jnp.exp(-ent)


def kernel(inputs, codebook):
    b, c, n = inputs.shape
    out_q, loss, perp, _hist = pl.pallas_call(
        _vq_body,
        grid=(b,),
        in_specs=[
            pl.BlockSpec((1, c, n), lambda i: (i, 0, 0)),
            pl.BlockSpec((_NUM_EMB, c), lambda i: (0, 0)),
        ],
        out_specs=[
            pl.BlockSpec((1, c, n), lambda i: (i, 0, 0)),
            pl.BlockSpec((1, 1), lambda i: (0, 0)),
            pl.BlockSpec((1, 1), lambda i: (0, 0)),
            pl.BlockSpec((_NUM_EMB, 1), lambda i: (0, 0)),
        ],
        out_shape=[
            jax.ShapeDtypeStruct((b, c, n), jnp.float32),
            jax.ShapeDtypeStruct((1, 1), jnp.float32),
            jax.ShapeDtypeStruct((1, 1), jnp.float32),
            jax.ShapeDtypeStruct((_NUM_EMB, 1), jnp.float32),
        ],
    )(inputs, codebook)
    return (loss[0, 0], out_q, perp[0, 0])
